# Initial kernel scaffold; baseline (speedup 1.0000x reference)
import functools

import jax
import jax.numpy as jnp
from jax import lax
from jax.experimental import pallas as pl
from jax.experimental.pallas import tpu as pltpu
from jax.experimental.pallas import tpu_sc as plsc

D = 64
SEQ = 200
BATCH = 4096
N = BATCH * SEQ
C = 128
NCHUNK = N // C

_info = plsc.get_sparse_core_info()
_NC = _info.num_cores
_NS = _info.num_subcores
NW = _NC * _NS
CPW = NCHUNK // NW


def _sc_body(q_hbm, t_hbm, wq_hbm, wt_hbm, wp2_hbm, out_hbm,
             qi_v, ti_v, qr_v, tr_v, ob_v, pos_v, sem):
    wid = lax.axis_index("s") * _NC + lax.axis_index("c")
    pltpu.sync_copy(wp2_hbm, pos_v)

    def chunk_body(k, carry):
        c = wid * CPW + k
        base = c * C
        pltpu.sync_copy(q_hbm.at[pl.ds(base, C)], qi_v)
        pltpu.sync_copy(t_hbm.at[pl.ds(base, C)], ti_v)
        cpq = pltpu.async_copy(wq_hbm.at[qi_v], qr_v, sem)
        cpt = pltpu.async_copy(wt_hbm.at[ti_v], tr_v, sem)
        cpq.wait()
        cpt.wait()
        base_s = lax.rem(base, SEQ)

        def row_body(i, carry2):
            for g in range(D // 16):
                sl = pl.ds(g * 16, 16)
                ob_v[i, sl] = qr_v[i, sl] + tr_v[i, sl] + pos_v[base_s + i, sl]
            return carry2

        lax.fori_loop(0, C, row_body, 0)
        pltpu.sync_copy(ob_v, out_hbm.at[pl.ds(base, C)])
        return carry

    lax.fori_loop(0, CPW, chunk_body, 0)


@jax.jit
def _sc_call(q, t, wq, wt, wp2):
    mesh = plsc.VectorSubcoreMesh(core_axis_name="c", subcore_axis_name="s")
    run = pl.kernel(
        _sc_body,
        out_type=jax.ShapeDtypeStruct((N, D), jnp.float32),
        mesh=mesh,
        scratch_types=[
            pltpu.VMEM((C,), jnp.int32),
            pltpu.VMEM((C,), jnp.int32),
            pltpu.VMEM((C, D), jnp.float32),
            pltpu.VMEM((C, D), jnp.float32),
            pltpu.VMEM((C, D), jnp.float32),
            pltpu.VMEM((2 * SEQ, D), jnp.float32),
            pltpu.SemaphoreType.DMA,
        ],
    )
    return run(q, t, wq, wt, wp2)


def kernel(questions, tags, W_question, W_tag, W_pos):
    q = questions.reshape(-1).astype(jnp.int32)
    t = tags.reshape(-1).astype(jnp.int32)
    wp2 = jnp.concatenate([W_pos, W_pos], axis=0)
    out = _sc_call(q, t, W_question, W_tag, wp2)
    return out.reshape(BATCH, SEQ, D)


# speedup vs baseline: 3.5307x; 3.5307x over previous
import functools

import jax
import jax.numpy as jnp
from jax import lax
from jax.experimental import pallas as pl
from jax.experimental.pallas import tpu as pltpu
from jax.experimental.pallas import tpu_sc as plsc

D = 64
SEQ = 200
BATCH = 4096
N = BATCH * SEQ
C = 128
NCHUNK = N // C

_info = plsc.get_sparse_core_info()
_NC = _info.num_cores
_NS = _info.num_subcores
NW = _NC * _NS
CPW = NCHUNK // NW


def _sc_body(q_hbm, t_hbm, wq_hbm, wt_hbm, wp2_hbm, out_hbm,
             qi_v, ti_v, qr_v, tr_v, ob_v, pos_v, sem):
    wid = lax.axis_index("s") * _NC + lax.axis_index("c")
    pltpu.sync_copy(wp2_hbm, pos_v)

    def chunk_body(k, carry):
        c = wid * CPW + k
        base = c * C
        pltpu.sync_copy(q_hbm.at[pl.ds(base, C)], qi_v)
        pltpu.sync_copy(t_hbm.at[pl.ds(base, C)], ti_v)
        cpq = pltpu.async_copy(wq_hbm.at[qi_v], qr_v, sem)
        cpt = pltpu.async_copy(wt_hbm.at[ti_v], tr_v, sem)
        cpq.wait()
        cpt.wait()
        base_s = lax.rem(base, SEQ)

        def row_body(i, carry2):
            for g in range(D // 16):
                sl = pl.ds(g * 16, 16)
                ob_v[i, sl] = qr_v[i, sl] + tr_v[i, sl] + pos_v[base_s + i, sl]
            return carry2

        lax.fori_loop(0, C, row_body, 0)
        pltpu.sync_copy(ob_v, out_hbm.at[pl.ds(base, C)])
        return carry

    lax.fori_loop(0, CPW, chunk_body, 0)


@jax.jit
def _sc_call(q, t, wq, wt, wp2):
    mesh = plsc.VectorSubcoreMesh(core_axis_name="c", subcore_axis_name="s")
    run = pl.kernel(
        _sc_body,
        out_type=jax.ShapeDtypeStruct((N, D), jnp.float32),
        mesh=mesh,
        scratch_types=[
            pltpu.VMEM((C,), jnp.int32),
            pltpu.VMEM((C,), jnp.int32),
            pltpu.VMEM((C, D), jnp.float32),
            pltpu.VMEM((C, D), jnp.float32),
            pltpu.VMEM((C, D), jnp.float32),
            pltpu.VMEM((2 * SEQ, D), jnp.float32),
            pltpu.SemaphoreType.DMA,
        ],
        compiler_params=pltpu.CompilerParams(use_tc_tiling_on_sc=False),
    )
    return run(q, t, wq, wt, wp2)


def kernel(questions, tags, W_question, W_tag, W_pos):
    q = questions.reshape(-1).astype(jnp.int32)
    t = tags.reshape(-1).astype(jnp.int32)
    wp2 = jnp.concatenate([W_pos, W_pos], axis=0)
    out = _sc_call(q, t, W_question, W_tag, wp2)
    return out.reshape(BATCH, SEQ, D)


# speedup vs baseline: 4.9524x; 1.4027x over previous
import functools

import jax
import jax.numpy as jnp
from jax import lax
from jax.experimental import pallas as pl
from jax.experimental.pallas import tpu as pltpu
from jax.experimental.pallas import tpu_sc as plsc

D = 64
SEQ = 200
BATCH = 4096
N = BATCH * SEQ
C = 128
CW = C * D
NCHUNK = N // C
NB = 2

_info = plsc.get_sparse_core_info()
_NC = _info.num_cores
_NS = _info.num_subcores
NW = _NC * _NS
CPW = NCHUNK // NW
UNROLL = 4


def _sc_body(q_hbm, t_hbm, wq_hbm, wt_hbm, wp2_hbm, out_hbm,
             qi, ti, ob, tr, pos_v, isem, gsem, osem):
    wid = lax.axis_index("s") * _NC + lax.axis_index("c")
    first = wid * CPW
    last = first + CPW - 1

    pltpu.sync_copy(wp2_hbm, pos_v)

    def idx_base(k):
        return jnp.minimum(k, last) * C

    def fire_idx(k, b):
        base = idx_base(k)
        pltpu.async_copy(q_hbm.at[pl.ds(base, C)], qi[b], isem[b])
        pltpu.async_copy(t_hbm.at[pl.ds(base, C)], ti[b], isem[b])

    def wait_idx(b):
        pltpu.make_async_copy(q_hbm.at[pl.ds(0, C)], qi[b], isem[b]).wait()
        pltpu.make_async_copy(t_hbm.at[pl.ds(0, C)], ti[b], isem[b]).wait()

    def fire_gathers(b):
        pltpu.async_copy(wq_hbm.at[qi[b]], ob[b], gsem[b])
        pltpu.async_copy(wt_hbm.at[ti[b]], tr[b], gsem[b])

    def wait_gathers(b):
        pltpu.make_async_copy(wq_hbm.at[qi[b]], ob[b], gsem[b]).wait()
        pltpu.make_async_copy(wt_hbm.at[ti[b]], tr[b], gsem[b]).wait()

    def fire_out(k, b):
        pltpu.async_copy(ob[b], out_hbm.at[pl.ds(k * C, C)], osem[b])

    def wait_out(b):
        pltpu.make_async_copy(ob[b], out_hbm.at[pl.ds(0, C)], osem[b]).wait()

    def compute(k, b):
        poff = lax.rem(k * C, SEQ) * D

        def sweep(ii, carry):
            for r in range(UNROLL):
                i = ii * UNROLL + r
                row_t = tr[b].at[i]
                row_o = ob[b].at[i]
                pbase = poff + i * D
                for g in range(D // 16):
                    sl = pl.ds(g * 16, 16)
                    x = row_t[sl] + pos_v[pl.ds(pbase + g * 16, 16)]
                    plsc.addupdate(row_o.at[sl], x)
            return carry

        lax.fori_loop(0, C // UNROLL, sweep, 0)

    def step(k, b, wait_prev_out):
        nb = (b + 1) % NB
        wait_idx(nb)
        if wait_prev_out:
            wait_out(nb)
        fire_gathers(nb)
        wait_gathers(b)
        fire_idx(k + NB, b)
        compute(k, b)
        fire_out(k, b)

    pltpu.sync_copy(q_hbm.at[pl.ds(first * C, C)], qi[0])
    pltpu.sync_copy(t_hbm.at[pl.ds(first * C, C)], ti[0])
    fire_gathers(0)
    fire_idx(first + 1, 1)

    step(first, 0, wait_prev_out=False)

    def loop_body(kk, carry):
        k = first + 1 + kk * NB
        for b in range(NB):
            step(k + b, (1 + b) % NB, wait_prev_out=True)
        return carry

    n_main = (CPW - 1) // NB
    lax.fori_loop(0, n_main, loop_body, 0)
    tail_start = first + 1 + n_main * NB
    for i in range(CPW - 1 - n_main * NB):
        step(tail_start + i, (1 + i) % NB, wait_prev_out=True)

    b_last = (CPW - 1) % NB
    nb_last = (b_last + 1) % NB
    wait_idx(b_last)
    wait_gathers(nb_last)
    for i in range(NB - 1):
        wait_out((b_last - i) % NB)


@jax.jit
def _sc_call(q, t, wq, wt, wp2):
    mesh = plsc.VectorSubcoreMesh(core_axis_name="c", subcore_axis_name="s")
    run = pl.kernel(
        _sc_body,
        out_type=jax.ShapeDtypeStruct((N, D), jnp.float32),
        mesh=mesh,
        scratch_types=[
            [pltpu.VMEM((C,), jnp.int32) for _ in range(NB)],
            [pltpu.VMEM((C,), jnp.int32) for _ in range(NB)],
            [pltpu.VMEM((C, D), jnp.float32) for _ in range(NB)],
            [pltpu.VMEM((C, D), jnp.float32) for _ in range(NB)],
            pltpu.VMEM((2 * SEQ * D,), jnp.float32),
            [pltpu.SemaphoreType.DMA for _ in range(NB)],
            [pltpu.SemaphoreType.DMA for _ in range(NB)],
            [pltpu.SemaphoreType.DMA for _ in range(NB)],
        ],
        compiler_params=pltpu.CompilerParams(use_tc_tiling_on_sc=False),
    )
    return run(q, t, wq, wt, wp2)


def kernel(questions, tags, W_question, W_tag, W_pos):
    q = questions.reshape(-1).astype(jnp.int32)
    t = tags.reshape(-1).astype(jnp.int32)
    wp2 = jnp.concatenate([W_pos, W_pos], axis=0).reshape(-1)
    out = _sc_call(q, t, W_question, W_tag, wp2)
    return out.reshape(BATCH, SEQ, D)


# speedup vs baseline: 6.5720x; 1.3270x over previous
import functools

import jax
import jax.numpy as jnp
from jax import lax
from jax.experimental import pallas as pl
from jax.experimental.pallas import tpu as pltpu
from jax.experimental.pallas import tpu_sc as plsc

D = 64
SEQ = 200
BATCH = 4096
G1 = 128
G2 = SEQ - G1
NB = 2

_info = plsc.get_sparse_core_info()
_NC = _info.num_cores
_NS = _info.num_subcores
NW = _NC * _NS
RPW = BATCH // NW
UNROLL = 4


def _sc_body(q_hbm, t_hbm, wq_hbm, wt_hbm, wp_hbm, out_hbm,
             qi, ti, ob, tr, pos_v, isem, gsem, osem):
    wid = lax.axis_index("s") * _NC + lax.axis_index("c")
    first = wid * RPW
    last = first + RPW - 1

    pltpu.sync_copy(wp_hbm, pos_v)

    def fire_idx(k, b):
        kc = jnp.minimum(k, last)
        pltpu.async_copy(q_hbm.at[kc], qi[b], isem[b])
        pltpu.async_copy(t_hbm.at[kc], ti[b], isem[b])

    def wait_idx(b):
        pltpu.make_async_copy(q_hbm.at[0], qi[b], isem[b]).wait()
        pltpu.make_async_copy(t_hbm.at[0], ti[b], isem[b]).wait()

    def fire_gathers(b):
        pltpu.async_copy(wq_hbm.at[qi[b].at[pl.ds(0, G1)]],
                         ob[b].at[pl.ds(0, G1)], gsem[b])
        pltpu.async_copy(wq_hbm.at[qi[b].at[pl.ds(G1, G2)]],
                         ob[b].at[pl.ds(G1, G2)], gsem[b])
        pltpu.async_copy(wt_hbm.at[ti[b].at[pl.ds(0, G1)]],
                         tr[b].at[pl.ds(0, G1)], gsem[b])
        pltpu.async_copy(wt_hbm.at[ti[b].at[pl.ds(G1, G2)]],
                         tr[b].at[pl.ds(G1, G2)], gsem[b])

    def wait_gathers(b):
        pltpu.make_async_copy(wq_hbm.at[qi[b].at[pl.ds(0, G1)]],
                              ob[b].at[pl.ds(0, G1)], gsem[b]).wait()
        pltpu.make_async_copy(wq_hbm.at[qi[b].at[pl.ds(G1, G2)]],
                              ob[b].at[pl.ds(G1, G2)], gsem[b]).wait()
        pltpu.make_async_copy(wt_hbm.at[ti[b].at[pl.ds(0, G1)]],
                              tr[b].at[pl.ds(0, G1)], gsem[b]).wait()
        pltpu.make_async_copy(wt_hbm.at[ti[b].at[pl.ds(G1, G2)]],
                              tr[b].at[pl.ds(G1, G2)], gsem[b]).wait()

    def fire_out(k, b):
        pltpu.async_copy(ob[b], out_hbm.at[k], osem[b])

    def wait_out(b):
        pltpu.make_async_copy(ob[b], out_hbm.at[0], osem[b]).wait()

    def compute(b):
        def sweep(ii, carry):
            for r in range(UNROLL):
                i = ii * UNROLL + r
                row_t = tr[b].at[i]
                row_o = ob[b].at[i]
                for g in range(D // 16):
                    sl = pl.ds(g * 16, 16)
                    x = row_t[sl] + pos_v[pl.ds(i * D + g * 16, 16)]
                    plsc.addupdate(row_o.at[sl], x)
            return carry

        lax.fori_loop(0, SEQ // UNROLL, sweep, 0)

    def step(k, b, wait_prev_out):
        nb = (b + 1) % NB
        wait_idx(nb)
        if wait_prev_out:
            wait_out(nb)
        fire_gathers(nb)
        wait_gathers(b)
        fire_idx(k + NB, b)
        compute(b)
        fire_out(k, b)

    pltpu.sync_copy(q_hbm.at[first], qi[0])
    pltpu.sync_copy(t_hbm.at[first], ti[0])
    fire_gathers(0)
    fire_idx(first + 1, 1)

    step(first, 0, wait_prev_out=False)

    def loop_body(kk, carry):
        k = first + 1 + kk * NB
        for b in range(NB):
            step(k + b, (1 + b) % NB, wait_prev_out=True)
        return carry

    n_main = (RPW - 1) // NB
    lax.fori_loop(0, n_main, loop_body, 0)
    tail_start = first + 1 + n_main * NB
    for i in range(RPW - 1 - n_main * NB):
        step(tail_start + i, (1 + i) % NB, wait_prev_out=True)

    b_last = (RPW - 1) % NB
    nb_last = (b_last + 1) % NB
    wait_idx(b_last)
    wait_gathers(nb_last)
    for i in range(NB - 1):
        wait_out((b_last - i) % NB)


@jax.jit
def _sc_call(q, t, wq, wt, wp):
    mesh = plsc.VectorSubcoreMesh(core_axis_name="c", subcore_axis_name="s")
    run = pl.kernel(
        _sc_body,
        out_type=jax.ShapeDtypeStruct((BATCH, SEQ, D), jnp.float32),
        mesh=mesh,
        scratch_types=[
            [pltpu.VMEM((SEQ,), jnp.int32) for _ in range(NB)],
            [pltpu.VMEM((SEQ,), jnp.int32) for _ in range(NB)],
            [pltpu.VMEM((SEQ, D), jnp.float32) for _ in range(NB)],
            [pltpu.VMEM((SEQ, D), jnp.float32) for _ in range(NB)],
            pltpu.VMEM((SEQ * D,), jnp.float32),
            [pltpu.SemaphoreType.DMA for _ in range(NB)],
            [pltpu.SemaphoreType.DMA for _ in range(NB)],
            [pltpu.SemaphoreType.DMA for _ in range(NB)],
        ],
        compiler_params=pltpu.CompilerParams(use_tc_tiling_on_sc=False),
    )
    return run(q, t, wq, wt, wp)


def kernel(questions, tags, W_question, W_tag, W_pos):
    return _sc_call(questions.astype(jnp.int32), tags.astype(jnp.int32),
                    W_question, W_tag, W_pos.reshape(-1))
